# baseline (speedup 1.0000x reference)
import jax
import jax.numpy as jnp
import numpy as np
from jax.experimental import pallas as pl
from jax.experimental.pallas import tpu as pltpu

_N, _D = 8, 2048
_KB = 512
_NK = _D // _KB
_NEG = -1e30
_BIG = 2 ** 30


def _make_gumbel() -> np.ndarray:
    u = jax.random.uniform(jax.random.key(42), (_N, _D), dtype=jnp.float32)
    g = -jnp.log(-jnp.log(u + 1e-20) + 1e-20)
    return np.asarray(g)


_GUMBEL = _make_gumbel()


def _fused_body(x_ref, w_ref, b_ref, g_ref, act_ref, logp_ref,
                pmax, pidx, plog, ssum):
    k = pl.program_id(1)

    logits = jax.lax.dot_general(
        x_ref[0], w_ref[0],
        dimension_numbers=(((1,), (1,)), ((), ())),
        preferred_element_type=jnp.float32,
        precision=jax.lax.Precision.DEFAULT,
    ) + b_ref[0]
    pert = logits + g_ref[0]
    kvec = k * _KB + jax.lax.broadcasted_iota(jnp.int32, (1, _KB), 1)

    @pl.when(k == 0)
    def _init():
        pmax[...] = jnp.full((1, _KB), _NEG, jnp.float32)
        pidx[...] = jnp.full((1, _KB), _BIG, jnp.int32)
        plog[...] = jnp.zeros((1, _KB), jnp.float32)
        ssum[...] = jnp.zeros((1, _KB), jnp.float32)

    upd = pert > pmax[...]
    pmax[...] = jnp.where(upd, pert, pmax[...])
    pidx[...] = jnp.where(upd, kvec, pidx[...])
    plog[...] = jnp.where(upd, logits, plog[...])
    ssum[...] = ssum[...] + jnp.exp(logits)

    @pl.when(k == _NK - 1)
    def _finalize():
        m = jnp.max(pmax[...], axis=1, keepdims=True)
        winidx = jnp.min(jnp.where(pmax[...] == m, pidx[...], _BIG),
                         axis=1, keepdims=True)
        blog = jnp.max(jnp.where(pidx[...] == winidx, plog[...], _NEG),
                       axis=1, keepdims=True)
        lse = jnp.log(jnp.sum(ssum[...], axis=1, keepdims=True))
        act_ref[0] = jnp.broadcast_to(winidx, (1, 128))
        logp_ref[0] = jnp.broadcast_to(blog - lse, (1, 128))


@jax.jit
def kernel(x, W, b):
    g = jnp.asarray(_GUMBEL)

    acts, logps = pl.pallas_call(
        _fused_body,
        grid=(_N, _NK),
        in_specs=[
            pl.BlockSpec((1, 1, _D), lambda n, k: (n, 0, 0)),
            pl.BlockSpec((1, _KB, _D), lambda n, k: (n, k, 0)),
            pl.BlockSpec((1, 1, _KB), lambda n, k: (n, 0, k)),
            pl.BlockSpec((1, 1, _KB), lambda n, k: (n, 0, k)),
        ],
        out_specs=[
            pl.BlockSpec((1, 1, 128), lambda n, k: (n, 0, 0)),
            pl.BlockSpec((1, 1, 128), lambda n, k: (n, 0, 0)),
        ],
        out_shape=[
            jax.ShapeDtypeStruct((_N, 1, 128), jnp.int32),
            jax.ShapeDtypeStruct((_N, 1, 128), jnp.float32),
        ],
        scratch_shapes=[
            pltpu.VMEM((1, _KB), jnp.float32),
            pltpu.VMEM((1, _KB), jnp.int32),
            pltpu.VMEM((1, _KB), jnp.float32),
            pltpu.VMEM((1, _KB), jnp.float32),
        ],
        compiler_params=pltpu.CompilerParams(
            dimension_semantics=("arbitrary", "arbitrary"),
        ),
    )(x[:, None, :], W, b[:, None, :], g[:, None, :])

    actions = acts[:, 0, :1].astype(jnp.int64)
    return actions, logps[:, 0, :1]


# speedup vs baseline: 1.1693x; 1.1693x over previous
import jax
import jax.numpy as jnp
import numpy as np
from jax.experimental import pallas as pl
from jax.experimental.pallas import tpu as pltpu

_N, _D = 8, 2048
_KB = 1024
_NK = _D // _KB
_NEG = -1e30
_BIG = 2 ** 30


def _make_gumbel() -> np.ndarray:
    u = jax.random.uniform(jax.random.key(42), (_N, _D), dtype=jnp.float32)
    g = -jnp.log(-jnp.log(u + 1e-20) + 1e-20)
    return np.asarray(g)


_GUMBEL = _make_gumbel()


def _fused_body(x_ref, w_ref, b_ref, g_ref, act_ref, logp_ref,
                pmax, pidx, plog, ssum):
    k = pl.program_id(1)

    logits = jax.lax.dot_general(
        x_ref[0], w_ref[0],
        dimension_numbers=(((1,), (1,)), ((), ())),
        preferred_element_type=jnp.float32,
        precision=jax.lax.Precision.DEFAULT,
    ) + b_ref[0]
    pert = logits + g_ref[0]
    kvec = k * _KB + jax.lax.broadcasted_iota(jnp.int32, (1, _KB), 1)

    @pl.when(k == 0)
    def _init():
        pmax[...] = jnp.full((1, _KB), _NEG, jnp.float32)
        pidx[...] = jnp.full((1, _KB), _BIG, jnp.int32)
        plog[...] = jnp.zeros((1, _KB), jnp.float32)
        ssum[...] = jnp.zeros((1, _KB), jnp.float32)

    upd = pert > pmax[...]
    pmax[...] = jnp.where(upd, pert, pmax[...])
    pidx[...] = jnp.where(upd, kvec, pidx[...])
    plog[...] = jnp.where(upd, logits, plog[...])
    ssum[...] = ssum[...] + jnp.exp(logits)

    @pl.when(k == _NK - 1)
    def _finalize():
        m = jnp.max(pmax[...], axis=1, keepdims=True)
        winidx = jnp.min(jnp.where(pmax[...] == m, pidx[...], _BIG),
                         axis=1, keepdims=True)
        blog = jnp.max(jnp.where(pidx[...] == winidx, plog[...], _NEG),
                       axis=1, keepdims=True)
        lse = jnp.log(jnp.sum(ssum[...], axis=1, keepdims=True))
        act_ref[0] = jnp.broadcast_to(winidx, (1, 128))
        logp_ref[0] = jnp.broadcast_to(blog - lse, (1, 128))


@jax.jit
def kernel(x, W, b):
    g = jnp.asarray(_GUMBEL)

    acts, logps = pl.pallas_call(
        _fused_body,
        grid=(_N, _NK),
        in_specs=[
            pl.BlockSpec((1, 1, _D), lambda n, k: (n, 0, 0)),
            pl.BlockSpec((1, _KB, _D), lambda n, k: (n, k, 0)),
            pl.BlockSpec((1, 1, _KB), lambda n, k: (n, 0, k)),
            pl.BlockSpec((1, 1, _KB), lambda n, k: (n, 0, k)),
        ],
        out_specs=[
            pl.BlockSpec((1, 1, 128), lambda n, k: (n, 0, 0)),
            pl.BlockSpec((1, 1, 128), lambda n, k: (n, 0, 0)),
        ],
        out_shape=[
            jax.ShapeDtypeStruct((_N, 1, 128), jnp.int32),
            jax.ShapeDtypeStruct((_N, 1, 128), jnp.float32),
        ],
        scratch_shapes=[
            pltpu.VMEM((1, _KB), jnp.float32),
            pltpu.VMEM((1, _KB), jnp.int32),
            pltpu.VMEM((1, _KB), jnp.float32),
            pltpu.VMEM((1, _KB), jnp.float32),
        ],
        compiler_params=pltpu.CompilerParams(
            dimension_semantics=("arbitrary", "arbitrary"),
        ),
    )(x[:, None, :], W, b[:, None, :], g[:, None, :])

    actions = acts[:, 0, :1].astype(jnp.int64)
    return actions, logps[:, 0, :1]
